# initial kernel scaffold (unmeasured)
import jax
import jax.numpy as jnp
from jax import lax
from jax.experimental import pallas as pl
from jax.experimental.pallas import tpu as pltpu


def kernel(
    t,
):
    def body(*refs):
        pass

    out_shape = jax.ShapeDtypeStruct(..., jnp.float32)
    return pl.pallas_call(body, out_shape=out_shape)(...)


# baseline (device time: 295801 ns/iter reference)
import jax
import jax.numpy as jnp
from jax import lax
from jax.experimental import pallas as pl
from jax.experimental.pallas import tpu as pltpu

N_DEV = 4


def kernel(t):
    m, n = t.shape
    chunk = m // N_DEV

    def body(t_ref, out_ref, recv_buf, send_buf,
             rs_send_sems, rs_recv_sems, ag_send_sems, ag_recv_sems):
        my = lax.axis_index("i")
        left = lax.rem(my - 1 + N_DEV, N_DEV)
        right = lax.rem(my + 1, N_DEV)

        barrier_sem = pltpu.get_barrier_semaphore()
        for nbr in (left, right):
            pl.semaphore_signal(
                barrier_sem, inc=1,
                device_id=(nbr,), device_id_type=pl.DeviceIdType.MESH,
            )
        pl.semaphore_wait(barrier_sem, 2)

        def t_chunk(c):
            return t_ref[pl.ds(c * chunk, chunk), :]

        rdma = pltpu.make_async_remote_copy(
            src_ref=t_ref.at[pl.ds(my * chunk, chunk), :],
            dst_ref=recv_buf.at[0],
            send_sem=rs_send_sems.at[0],
            recv_sem=rs_recv_sems.at[0],
            device_id=(right,),
            device_id_type=pl.DeviceIdType.MESH,
        )
        rdma.start()
        rdma.wait()

        for h in range(1, N_DEV - 1):
            c = lax.rem(my - h + N_DEV, N_DEV)
            send_buf[...] = recv_buf[h - 1] + t_chunk(c)
            rdma = pltpu.make_async_remote_copy(
                src_ref=send_buf,
                dst_ref=recv_buf.at[h],
                send_sem=rs_send_sems.at[h],
                recv_sem=rs_recv_sems.at[h],
                device_id=(right,),
                device_id_type=pl.DeviceIdType.MESH,
            )
            rdma.start()
            rdma.wait()

        own = lax.rem(my + 1, N_DEV)
        s = recv_buf[N_DEV - 2] + t_chunk(own)
        r = jnp.maximum(s, 0.0)
        out_ref[pl.ds(own * chunk, chunk), :] = jnp.tanh(s) * s * s + r * r * r

        for h in range(N_DEV - 1):
            c = lax.rem(my + 1 - h + N_DEV, N_DEV)
            rdma = pltpu.make_async_remote_copy(
                src_ref=out_ref.at[pl.ds(c * chunk, chunk), :],
                dst_ref=out_ref.at[pl.ds(c * chunk, chunk), :],
                send_sem=ag_send_sems.at[h],
                recv_sem=ag_recv_sems.at[h],
                device_id=(right,),
                device_id_type=pl.DeviceIdType.MESH,
            )
            rdma.start()
            rdma.wait()

    return pl.pallas_call(
        body,
        out_shape=jax.ShapeDtypeStruct((m, n), t.dtype),
        in_specs=[pl.BlockSpec(memory_space=pltpu.VMEM)],
        out_specs=pl.BlockSpec(memory_space=pltpu.VMEM),
        scratch_shapes=[
            pltpu.VMEM((N_DEV - 1, chunk, n), t.dtype),
            pltpu.VMEM((chunk, n), t.dtype),
            pltpu.SemaphoreType.DMA((N_DEV - 1,)),
            pltpu.SemaphoreType.DMA((N_DEV - 1,)),
            pltpu.SemaphoreType.DMA((N_DEV - 1,)),
            pltpu.SemaphoreType.DMA((N_DEV - 1,)),
        ],
        compiler_params=pltpu.CompilerParams(collective_id=0),
    )(t)


# device time: 157764 ns/iter; 1.8750x vs baseline; 1.8750x over previous
import jax
import jax.numpy as jnp
from jax import lax
from jax.experimental import pallas as pl
from jax.experimental.pallas import tpu as pltpu

N_DEV = 4


def kernel(t):
    m, n = t.shape
    half = m // 4
    quar = m // 8
    breg = m // 2

    def f(s):
        r = jnp.maximum(s, 0.0)
        return jnp.tanh(s) * s * s + r * r * r

    def body(t_ref, out_ref, bufA1, bufB1, bufA2, bufB2, ssems, rsems):
        p = lax.axis_index("i")
        xp = p ^ 1
        yp = 3 - p

        h1A = jnp.where((p == 1) | (p == 2), 1, 0)
        q2A = jnp.where(p >= 2, 1, 0)
        h1B = q2A
        q2B = p % 2
        gA = (2 * h1A + q2A) * quar
        gB = breg + (2 * h1B + q2B) * quar

        barrier_sem = pltpu.get_barrier_semaphore()
        for nbr in (xp, yp):
            pl.semaphore_signal(
                barrier_sem, inc=1,
                device_id=(nbr,), device_id_type=pl.DeviceIdType.MESH,
            )
        pl.semaphore_wait(barrier_sem, 2)

        def xchg(i, src, dst, partner):
            return pltpu.make_async_remote_copy(
                src_ref=src, dst_ref=dst,
                send_sem=ssems.at[i], recv_sem=rsems.at[i],
                device_id=(partner,), device_id_type=pl.DeviceIdType.MESH,
            )

        ra = xchg(0, t_ref.at[pl.ds((1 - h1A) * half, half), :], bufA1, xp)
        rb = xchg(1, t_ref.at[pl.ds(breg + (1 - h1B) * half, half), :],
                  bufB1, yp)
        ra.start()
        rb.start()
        ra.wait()
        bufA1[...] = bufA1[...] + t_ref[pl.ds(h1A * half, half), :]
        rb.wait()
        bufB1[...] = bufB1[...] + t_ref[pl.ds(breg + h1B * half, half), :]

        ra = xchg(2, bufA1.at[pl.ds((1 - q2A) * quar, quar), :], bufA2, yp)
        rb = xchg(3, bufB1.at[pl.ds((1 - q2B) * quar, quar), :], bufB2, xp)
        ra.start()
        rb.start()
        ra.wait()
        out_ref[pl.ds(gA, quar), :] = f(bufA1[pl.ds(q2A * quar, quar), :]
                                        + bufA2[...])
        rb.wait()
        out_ref[pl.ds(gB, quar), :] = f(bufB1[pl.ds(q2B * quar, quar), :]
                                        + bufB2[...])

        ra = xchg(4, out_ref.at[pl.ds(gA, quar), :],
                  out_ref.at[pl.ds(gA, quar), :], yp)
        rb = xchg(5, out_ref.at[pl.ds(gB, quar), :],
                  out_ref.at[pl.ds(gB, quar), :], xp)
        ra.start()
        rb.start()
        ra.wait()
        rb.wait()

        ra = xchg(6, out_ref.at[pl.ds(h1A * half, half), :],
                  out_ref.at[pl.ds(h1A * half, half), :], xp)
        rb = xchg(7, out_ref.at[pl.ds(breg + h1B * half, half), :],
                  out_ref.at[pl.ds(breg + h1B * half, half), :], yp)
        ra.start()
        rb.start()
        ra.wait()
        rb.wait()

    return pl.pallas_call(
        body,
        out_shape=jax.ShapeDtypeStruct((m, n), t.dtype),
        in_specs=[pl.BlockSpec(memory_space=pltpu.VMEM)],
        out_specs=pl.BlockSpec(memory_space=pltpu.VMEM),
        scratch_shapes=[
            pltpu.VMEM((half, n), t.dtype),
            pltpu.VMEM((half, n), t.dtype),
            pltpu.VMEM((quar, n), t.dtype),
            pltpu.VMEM((quar, n), t.dtype),
            pltpu.SemaphoreType.DMA((8,)),
            pltpu.SemaphoreType.DMA((8,)),
        ],
        compiler_params=pltpu.CompilerParams(collective_id=0),
    )(t)


# device time: 152856 ns/iter; 1.9352x vs baseline; 1.0321x over previous
import jax
import jax.numpy as jnp
from jax import lax
from jax.experimental import pallas as pl
from jax.experimental.pallas import tpu as pltpu

N_DEV = 4


def kernel(t):
    m, n = t.shape
    half = m // 4
    quar = m // 8
    breg = m // 2

    def f(s):
        r = jnp.maximum(s, 0.0)
        return jnp.tanh(s) * s * s + r * r * r

    def body(t_ref, out_ref, bufA1, bufB1, bufA2, bufB2, tA_loc, tB_loc,
             ssems, rsems, lsems):
        p = lax.axis_index("i")
        xp = p ^ 1
        yp = 3 - p

        h1A = jnp.where((p == 1) | (p == 2), 1, 0)
        q2A = jnp.where(p >= 2, 1, 0)
        h1B = q2A
        q2B = p % 2
        gA = (2 * h1A + q2A) * quar
        gB = breg + (2 * h1B + q2B) * quar

        cpA = pltpu.make_async_copy(
            t_ref.at[pl.ds(h1A * half, half), :], tA_loc, lsems.at[0])
        cpB = pltpu.make_async_copy(
            t_ref.at[pl.ds(breg + h1B * half, half), :], tB_loc, lsems.at[1])
        cpA.start()
        cpB.start()

        barrier_sem = pltpu.get_barrier_semaphore()
        for nbr in (xp, yp):
            pl.semaphore_signal(
                barrier_sem, inc=1,
                device_id=(nbr,), device_id_type=pl.DeviceIdType.MESH,
            )
        pl.semaphore_wait(barrier_sem, 2)

        def xchg(i, src, dst, partner):
            return pltpu.make_async_remote_copy(
                src_ref=src, dst_ref=dst,
                send_sem=ssems.at[i], recv_sem=rsems.at[i],
                device_id=(partner,), device_id_type=pl.DeviceIdType.MESH,
            )

        ra = xchg(0, t_ref.at[pl.ds((1 - h1A) * half, half), :], bufA1, xp)
        rb = xchg(1, t_ref.at[pl.ds(breg + (1 - h1B) * half, half), :],
                  bufB1, yp)
        ra.start()
        rb.start()
        cpA.wait()
        cpB.wait()
        ra.wait()
        bufA1[...] = bufA1[...] + tA_loc[...]
        rb.wait()
        bufB1[...] = bufB1[...] + tB_loc[...]

        ra = xchg(2, bufA1.at[pl.ds((1 - q2A) * quar, quar), :], bufA2, yp)
        rb = xchg(3, bufB1.at[pl.ds((1 - q2B) * quar, quar), :], bufB2, xp)
        ra.start()
        rb.start()
        ra.wait()
        out_ref[pl.ds(gA, quar), :] = f(bufA1[pl.ds(q2A * quar, quar), :]
                                        + bufA2[...])
        rb.wait()
        out_ref[pl.ds(gB, quar), :] = f(bufB1[pl.ds(q2B * quar, quar), :]
                                        + bufB2[...])

        ra = xchg(4, out_ref.at[pl.ds(gA, quar), :],
                  out_ref.at[pl.ds(gA, quar), :], yp)
        rb = xchg(5, out_ref.at[pl.ds(gB, quar), :],
                  out_ref.at[pl.ds(gB, quar), :], xp)
        ra.start()
        rb.start()
        ra.wait()
        rb.wait()

        ra = xchg(6, out_ref.at[pl.ds(h1A * half, half), :],
                  out_ref.at[pl.ds(h1A * half, half), :], xp)
        rb = xchg(7, out_ref.at[pl.ds(breg + h1B * half, half), :],
                  out_ref.at[pl.ds(breg + h1B * half, half), :], yp)
        ra.start()
        rb.start()
        ra.wait()
        rb.wait()

    return pl.pallas_call(
        body,
        out_shape=jax.ShapeDtypeStruct((m, n), t.dtype),
        in_specs=[pl.BlockSpec(memory_space=pl.ANY)],
        out_specs=pl.BlockSpec(memory_space=pltpu.VMEM),
        scratch_shapes=[
            pltpu.VMEM((half, n), t.dtype),
            pltpu.VMEM((half, n), t.dtype),
            pltpu.VMEM((quar, n), t.dtype),
            pltpu.VMEM((quar, n), t.dtype),
            pltpu.VMEM((half, n), t.dtype),
            pltpu.VMEM((half, n), t.dtype),
            pltpu.SemaphoreType.DMA((8,)),
            pltpu.SemaphoreType.DMA((8,)),
            pltpu.SemaphoreType.DMA((2,)),
        ],
        compiler_params=pltpu.CompilerParams(collective_id=0),
    )(t)
